# Initial kernel scaffold; baseline (speedup 1.0000x reference)
import functools

import jax
import jax.numpy as jnp
from jax import lax
from jax.experimental import pallas as pl
from jax.experimental.pallas import tpu as pltpu
from jax.experimental.pallas import tpu_sc as plsc

BATCH = 16384
SEQ = 50
DIM = 64
NV = DIM // 16
G = 8
NBUF = 2


def _body(prem_hbm, hyp_hbm, table_hbm, wvec_hbm, out_hbm,
          idxp_v, idxh_v, rows_v, w_v, out_v, sem0, sem1):
    info = plsc.get_sparse_core_info()
    nc = info.num_cores
    wid = lax.axis_index("s") * nc + lax.axis_index("c")
    bpw = BATCH // (nc * info.num_subcores)
    nchunk = bpw // G
    base = pl.multiple_of(wid * bpw, bpw)
    sems = (sem0, sem1)

    pltpu.sync_copy(wvec_hbm, w_v)

    def gather_copies(g, b):
        copies = []
        for j in range(G):
            copies.append(pltpu.make_async_copy(
                table_hbm.at[idxp_v.at[b, j]],
                rows_v.at[b, j, pl.ds(0, SEQ)], sems[b]))
            copies.append(pltpu.make_async_copy(
                table_hbm.at[idxh_v.at[b, j]],
                rows_v.at[b, j, pl.ds(SEQ, SEQ)], sems[b]))
        return copies

    def load_indices(g, b):
        row0 = base + g * G
        pltpu.sync_copy(prem_hbm.at[pl.ds(row0, G)], idxp_v.at[b])
        pltpu.sync_copy(hyp_hbm.at[pl.ds(row0, G)], idxh_v.at[b])

    def fire(g, b):
        for c in gather_copies(g, b):
            c.start()

    def drain(g, b):
        for c in gather_copies(g, b):
            c.wait()

    def compute(g, b):
        for j in range(G):
            def seq_body(s, acc):
                new = tuple(
                    jnp.maximum(acc[d], rows_v[b, j, s, pl.ds(d * 16, 16)])
                    for d in range(NV)
                ) + tuple(
                    jnp.maximum(acc[NV + d], rows_v[b, j, SEQ + s, pl.ds(d * 16, 16)])
                    for d in range(NV)
                )
                return new

            init = tuple(rows_v[b, j, 0, pl.ds(d * 16, 16)] for d in range(NV)) \
                 + tuple(rows_v[b, j, SEQ, pl.ds(d * 16, 16)] for d in range(NV))
            acc = lax.fori_loop(1, SEQ, seq_body, init)

            sv = acc[0] * w_v[pl.ds(0, 16)]
            for d in range(1, NV):
                sv = sv + acc[d] * w_v[pl.ds(d * 16, 16)]
            for d in range(NV):
                sv = sv + acc[NV + d] * w_v[pl.ds(DIM + d * 16, 16)]
            logit = jnp.sum(sv) + w_v[2 * DIM]
            out_v[g * G + j] = logit

    for b in range(NBUF):
        load_indices(b, b)
        fire(b, b)

    def outer(i, _):
        g0 = i * NBUF
        for b in range(NBUF):
            g = g0 + b
            drain(g, b)
            compute(g, b)

            @pl.when(g + NBUF < nchunk)
            def _():
                load_indices(g + NBUF, b)
                fire(g + NBUF, b)
        return 0

    lax.fori_loop(0, nchunk // NBUF, outer, 0)

    def sig_body(k, _):
        off = pl.multiple_of(k * 16, 16)
        x = out_v[pl.ds(off, 16)]
        out_v[pl.ds(off, 16)] = 1.0 / (1.0 + jnp.exp(-x))
        return 0

    lax.fori_loop(0, bpw // 16, sig_body, 0)
    pltpu.sync_copy(out_v, out_hbm.at[pl.ds(base, bpw)])


def kernel(premise, hypothesis, table, W, b):
    info = plsc.get_sparse_core_info()
    nw = info.num_cores * info.num_subcores
    bpw = BATCH // nw

    wvec = jnp.zeros((144,), jnp.float32)
    wvec = wvec.at[: 2 * DIM].set(W.reshape(-1)).at[2 * DIM].set(b[0])

    mesh = plsc.VectorSubcoreMesh(core_axis_name="c", subcore_axis_name="s")
    k = functools.partial(
        pl.kernel,
        out_type=jax.ShapeDtypeStruct((BATCH,), jnp.float32),
        mesh=mesh,
        scratch_types=[
            pltpu.VMEM((NBUF, G, SEQ), jnp.int32),
            pltpu.VMEM((NBUF, G, SEQ), jnp.int32),
            pltpu.VMEM((NBUF, G, 2 * SEQ, DIM), jnp.float32),
            pltpu.VMEM((144,), jnp.float32),
            pltpu.VMEM((bpw,), jnp.float32),
            pltpu.SemaphoreType.DMA,
            pltpu.SemaphoreType.DMA,
        ],
    )(_body)
    return k(premise.astype(jnp.int32), hypothesis.astype(jnp.int32),
             table, wvec)


# speedup vs baseline: 4.5765x; 4.5765x over previous
import functools

import numpy as np
import jax
import jax.numpy as jnp
from jax import lax
from jax.experimental import pallas as pl
from jax.experimental.pallas import tpu as pltpu
from jax.experimental.pallas import tpu_sc as plsc

BATCH = 16384
SEQ = 50
DIM = 64
NV = DIM // 16
G = 8
NBUF = 2


def _body(prem_hbm, hyp_hbm, table_hbm, wvec_hbm, out_hbm,
          idxp_v, idxh_v, rows_v, w_v, dot_v, out_v, sem0, sem1):
    info = plsc.get_sparse_core_info()
    nc = info.num_cores
    wid = lax.axis_index("s") * nc + lax.axis_index("c")
    bpw = BATCH // (nc * info.num_subcores)
    nchunk = bpw // G
    base = pl.multiple_of(wid * bpw, bpw)
    sems = (sem0, sem1)

    pltpu.sync_copy(wvec_hbm, w_v)
    bias = w_v[pl.ds(2 * DIM, 16)][0]
    lane = lax.iota(jnp.int32, 16)
    lane_lo = lane & 7
    store_mask = lane < 8
    col_base = lane_lo * 16

    def gather_copies(g, b):
        copies = []
        for j in range(G):
            roff = (b * G + j) * 2 * SEQ
            copies.append(pltpu.make_async_copy(
                table_hbm.at[idxp_v.at[b, j]],
                rows_v.at[pl.ds(roff, SEQ)], sems[b]))
            copies.append(pltpu.make_async_copy(
                table_hbm.at[idxh_v.at[b, j]],
                rows_v.at[pl.ds(roff + SEQ, SEQ)], sems[b]))
        return copies

    def load_indices(g, b):
        row0 = base + g * G
        pltpu.sync_copy(prem_hbm.at[pl.ds(row0, G)], idxp_v.at[b])
        pltpu.sync_copy(hyp_hbm.at[pl.ds(row0, G)], idxh_v.at[b])

    def fire(g, b):
        for c in gather_copies(g, b):
            c.start()

    def drain(g, b):
        for c in gather_copies(g, b):
            c.wait()

    lane_d = [lane + d * 16 for d in range(NV)]
    neg_inf = jnp.full((16,), -jnp.inf, jnp.float32)

    def compute(g, b):
        for j in range(G):
            roff = (b * G + j) * 2 * SEQ

            def seq_body(s, acc):
                sp = jnp.full((16,), roff + s, jnp.int32)
                sh = jnp.full((16,), roff + SEQ + s, jnp.int32)
                new = tuple(
                    jnp.maximum(acc[d],
                                plsc.load_gather(rows_v, [sp, lane_d[d]]))
                    for d in range(NV)
                ) + tuple(
                    jnp.maximum(acc[NV + d],
                                plsc.load_gather(rows_v, [sh, lane_d[d]]))
                    for d in range(NV)
                )
                return new

            acc = lax.fori_loop(0, SEQ, seq_body, (neg_inf,) * (2 * NV))

            sv = acc[0] * w_v[pl.ds(0, 16)]
            for d in range(1, NV):
                sv = sv + acc[d] * w_v[pl.ds(d * 16, 16)]
            for d in range(NV):
                sv = sv + acc[NV + d] * w_v[pl.ds(DIM + d * 16, 16)]
            dot_v[pl.ds(j * 16, 16)] = sv

        tot = plsc.load_gather(dot_v, [col_base])
        for l in range(1, 16):
            tot = tot + plsc.load_gather(dot_v, [col_base + l])
        tot = tot + bias
        plsc.store_scatter(out_v, [g * G + lane_lo], tot, mask=store_mask)

    for b in range(NBUF):
        load_indices(b, b)
        fire(b, b)

    def outer(i, _):
        g0 = i * NBUF
        for b in range(NBUF):
            g = g0 + b
            drain(g, b)
            compute(g, b)

            @pl.when(g + NBUF < nchunk)
            def _():
                load_indices(g + NBUF, b)
                fire(g + NBUF, b)
        return 0

    lax.fori_loop(0, nchunk // NBUF, outer, 0)

    def sig_body(k, _):
        iv = k * 16 + lane
        x = plsc.load_gather(out_v, [iv])
        plsc.store_scatter(out_v, [iv], 1.0 / (1.0 + jnp.exp(-x)))
        return 0

    lax.fori_loop(0, bpw // 16, sig_body, 0)
    pltpu.sync_copy(out_v, out_hbm.at[pl.ds(base, bpw)])


def kernel(premise, hypothesis, table, W, b):
    info = plsc.get_sparse_core_info()
    nw = info.num_cores * info.num_subcores
    bpw = BATCH // nw

    wvec = jnp.zeros((144,), jnp.float32)
    wvec = wvec.at[: 2 * DIM].set(W.reshape(-1)).at[2 * DIM].set(b[0])

    mesh = plsc.VectorSubcoreMesh(core_axis_name="c", subcore_axis_name="s")
    k = functools.partial(
        pl.kernel,
        out_type=jax.ShapeDtypeStruct((BATCH,), jnp.float32),
        mesh=mesh,
        compiler_params=pltpu.CompilerParams(
            needs_layout_passes=False, use_tc_tiling_on_sc=False),
        scratch_types=[
            pltpu.VMEM((NBUF, G, SEQ), jnp.int32),
            pltpu.VMEM((NBUF, G, SEQ), jnp.int32),
            pltpu.VMEM((NBUF * G * 2 * SEQ, DIM), jnp.float32),
            pltpu.VMEM((144,), jnp.float32),
            pltpu.VMEM((G * 16,), jnp.float32),
            pltpu.VMEM((bpw,), jnp.float32),
            pltpu.SemaphoreType.DMA,
            pltpu.SemaphoreType.DMA,
        ],
    )(_body)
    return k(premise.astype(jnp.int32), hypothesis.astype(jnp.int32),
             table, wvec)
